# Initial kernel scaffold; baseline (speedup 1.0000x reference)
import jax
import jax.numpy as jnp
from jax.experimental import pallas as pl


def kernel(input, token_embedding_table):
    raise NotImplementedError("write your pallas kernel here")


# speedup vs baseline: 1.0349x; 1.0349x over previous
import functools

import jax
import jax.numpy as jnp
from jax import lax
from jax.experimental import pallas as pl
from jax.experimental.pallas import tpu as pltpu
from jax.experimental.pallas import tpu_sc as plsc

_B = 51200
_D = 1000
_NC = 2
_NS = 16
_NW = _NC * _NS
_BPW = _B // _NW
_CH = 40
_NCH = _BPW // _CH


def _gather_rows(table, idx_flat):
    mesh = plsc.VectorSubcoreMesh(core_axis_name="c", subcore_axis_name="s")

    @functools.partial(
        pl.kernel,
        out_type=jax.ShapeDtypeStruct((_B, _D), jnp.float32),
        mesh=mesh,
        compiler_params=pltpu.CompilerParams(use_tc_tiling_on_sc=False),
        scratch_types=[
            pltpu.VMEM((_BPW,), jnp.int32),
            pltpu.VMEM((_CH, _D), jnp.float32),
            pltpu.VMEM((_CH, _D), jnp.float32),
            pltpu.SemaphoreType.DMA,
            pltpu.SemaphoreType.DMA,
        ],
    )
    def k(table_hbm, idx_hbm, out_hbm, idx_v, buf0, buf1, sem0, sem1):
        wid = lax.axis_index("s") * _NC + lax.axis_index("c")
        base = wid * _BPW
        pltpu.sync_copy(idx_hbm.at[pl.ds(base, _BPW)], idx_v)

        def gather_start(j, buf, sem):
            off = pl.multiple_of(j * _CH, 8)
            pltpu.async_copy(table_hbm.at[idx_v.at[pl.ds(off, _CH)]], buf, sem)

        def drain(j, buf, sem):
            pltpu.make_async_copy(
                table_hbm.at[idx_v.at[pl.ds(0, _CH)]], buf, sem
            ).wait()
            pltpu.sync_copy(buf, out_hbm.at[pl.ds(base + j * _CH, _CH)])

        gather_start(0, buf0, sem0)
        gather_start(1, buf1, sem1)

        def body(j2, _):
            j = j2 * 2
            drain(j, buf0, sem0)

            @pl.when(j + 2 < _NCH)
            def _():
                gather_start(j + 2, buf0, sem0)

            drain(j + 1, buf1, sem1)

            @pl.when(j + 3 < _NCH)
            def _():
                gather_start(j + 3, buf1, sem1)

            return 0

        lax.fori_loop(0, _NCH // 2, body, 0)

    return k(table, idx_flat)


def kernel(input, token_embedding_table):
    idx_flat = input.reshape(_B).astype(jnp.int32)
    out = _gather_rows(token_embedding_table, idx_flat)
    return out.reshape(input.shape + (_D,))
